# baseline (speedup 1.0000x reference)
import functools

import jax
import jax.numpy as jnp
from jax import lax
from jax.experimental import pallas as pl
from jax.experimental.pallas import tpu as pltpu
from jax.experimental.pallas import tpu_sc as plsc

B, S, D = 4, 8192, 1024
NC, NS = 2, 16
NW = NC * NS
ROWS = B * S
RPW = ROWS // NW
CR = 64
CC = 256
NCC = D // CC
NCHUNK = (RPW // CR) * NCC
NBUF = 3
VEC = 16
UNROLL = 16


def kernel(inputs, pos_table):
    x2 = inputs.reshape(ROWS, D)
    mesh = plsc.VectorSubcoreMesh(core_axis_name="c", subcore_axis_name="s")

    @functools.partial(
        pl.kernel,
        mesh=mesh,
        out_type=jax.ShapeDtypeStruct((ROWS, D), jnp.float32),
        scratch_types=(
            [pltpu.VMEM((CR, CC), jnp.float32) for _ in range(2 * NBUF)]
            + [pltpu.SemaphoreType.DMA((NBUF,)), pltpu.SemaphoreType.DMA((NBUF,))]
        ),
    )
    def sc_add(x_hbm, t_hbm, o_hbm, *scratch):
        xb = scratch[:NBUF]
        ob = scratch[NBUF:2 * NBUF]
        in_sem, out_sem = scratch[2 * NBUF], scratch[2 * NBUF + 1]

        wid = lax.axis_index("s") * NC + lax.axis_index("c")
        row0 = wid * RPW
        trow0 = row0 % S

        def offs(c):
            r = (c // NCC) * CR
            col = (c % NCC) * CC
            return r, col

        def issue_in(c, b):
            r, col = offs(c)
            pltpu.async_copy(x_hbm.at[pl.ds(row0 + r, CR), pl.ds(col, CC)],
                             xb[b], in_sem.at[b])
            pltpu.async_copy(t_hbm.at[pl.ds(trow0 + r, CR), pl.ds(col, CC)],
                             ob[b], in_sem.at[b])

        def wait_in(c, b):
            r, col = offs(c)
            pltpu.make_async_copy(x_hbm.at[pl.ds(row0 + r, CR), pl.ds(col, CC)],
                                  xb[b], in_sem.at[b]).wait()
            pltpu.make_async_copy(t_hbm.at[pl.ds(trow0 + r, CR), pl.ds(col, CC)],
                                  ob[b], in_sem.at[b]).wait()

        def issue_out(c, b):
            r, col = offs(c)
            pltpu.async_copy(ob[b], o_hbm.at[pl.ds(row0 + r, CR), pl.ds(col, CC)],
                             out_sem.at[b])

        def wait_out(c, b):
            r, col = offs(c)
            pltpu.make_async_copy(ob[b], o_hbm.at[pl.ds(row0 + r, CR), pl.ds(col, CC)],
                                  out_sem.at[b]).wait()

        def compute(b):
            def kbody(k, carry):
                for u in range(UNROLL):
                    idx = k * UNROLL + u
                    r = idx // (CC // VEC)
                    col = (idx % (CC // VEC)) * VEC
                    plsc.addupdate(ob[b].at[r, pl.ds(col, VEC)],
                                   xb[b][r, pl.ds(col, VEC)])
                return carry
            lax.fori_loop(0, (CR * CC) // (VEC * UNROLL), kbody, 0)

        def step(c, b):
            wait_in(c, b)
            compute(b)
            issue_out(c, b)


        for b in range(NBUF):
            issue_in(b, b)

        for c in range(NBUF):
            step(c, c)
            if c >= 1:
                wait_out(c - 1, (c - 1) % NBUF)
                if c - 1 + NBUF < NCHUNK:
                    issue_in(c - 1 + NBUF, (c - 1) % NBUF)

        NSTEADY = (NCHUNK - 2 * NBUF + 1) // NBUF

        def gbody(g, carry):
            for b in range(NBUF):
                c = g * NBUF + b
                step(c, b)
                wait_out(c - 1, (b + NBUF - 1) % NBUF)
                issue_in(c - 1 + NBUF, (b + NBUF - 1) % NBUF)
            return carry

        lax.fori_loop(1, NSTEADY + 1, gbody, 0)

        for c in range((NSTEADY + 1) * NBUF, NCHUNK):
            step(c, c % NBUF)
            wait_out(c - 1, (c - 1) % NBUF)
            if c - 1 + NBUF < NCHUNK:
                issue_in(c - 1 + NBUF, (c - 1) % NBUF)

        wait_out(NCHUNK - 1, (NCHUNK - 1) % NBUF)

    return sc_add(x2, pos_table).reshape(B, S, D)


# speedup vs baseline: 1.5522x; 1.5522x over previous
import functools

import jax
import jax.numpy as jnp
from jax import lax
from jax.experimental import pallas as pl
from jax.experimental.pallas import tpu as pltpu
from jax.experimental.pallas import tpu_sc as plsc

B, S, D = 4, 8192, 1024
NC, NS = 2, 16
NW = NC * NS
ROWS = B * S
RPW = ROWS // NW
CR = 128
CC = 128
NCC = D // CC
NCHUNK = (RPW // CR) * NCC
NBUF = 3
VEC = 16
UNROLL = 16


def kernel(inputs, pos_table):
    x2 = inputs.reshape(ROWS, D)
    mesh = plsc.VectorSubcoreMesh(core_axis_name="c", subcore_axis_name="s")

    @functools.partial(
        pl.kernel,
        mesh=mesh,
        out_type=jax.ShapeDtypeStruct((ROWS, D), jnp.float32),
        scratch_types=(
            [pltpu.VMEM((CR, CC), jnp.float32) for _ in range(2 * NBUF)]
            + [pltpu.SemaphoreType.DMA((NBUF,)), pltpu.SemaphoreType.DMA((NBUF,))]
        ),
    )
    def sc_add(x_hbm, t_hbm, o_hbm, *scratch):
        xb = scratch[:NBUF]
        ob = scratch[NBUF:2 * NBUF]
        in_sem, out_sem = scratch[2 * NBUF], scratch[2 * NBUF + 1]

        wid = lax.axis_index("s") * NC + lax.axis_index("c")
        row0 = wid * RPW
        trow0 = row0 % S

        def offs(c):
            r = (c // NCC) * CR
            col = (c % NCC) * CC
            return r, col

        def issue_in(c, b):
            r, col = offs(c)
            pltpu.async_copy(x_hbm.at[pl.ds(row0 + r, CR), pl.ds(col, CC)],
                             xb[b], in_sem.at[b])
            pltpu.async_copy(t_hbm.at[pl.ds(trow0 + r, CR), pl.ds(col, CC)],
                             ob[b], in_sem.at[b])

        def wait_in(c, b):
            r, col = offs(c)
            pltpu.make_async_copy(x_hbm.at[pl.ds(row0 + r, CR), pl.ds(col, CC)],
                                  xb[b], in_sem.at[b]).wait()
            pltpu.make_async_copy(t_hbm.at[pl.ds(trow0 + r, CR), pl.ds(col, CC)],
                                  ob[b], in_sem.at[b]).wait()

        def issue_out(c, b):
            r, col = offs(c)
            pltpu.async_copy(ob[b], o_hbm.at[pl.ds(row0 + r, CR), pl.ds(col, CC)],
                             out_sem.at[b])

        def wait_out(c, b):
            r, col = offs(c)
            pltpu.make_async_copy(ob[b], o_hbm.at[pl.ds(row0 + r, CR), pl.ds(col, CC)],
                                  out_sem.at[b]).wait()

        def compute(b):
            def kbody(k, carry):
                for u in range(UNROLL):
                    idx = k * UNROLL + u
                    r = idx // (CC // VEC)
                    col = (idx % (CC // VEC)) * VEC
                    plsc.addupdate(ob[b].at[r, pl.ds(col, VEC)],
                                   xb[b][r, pl.ds(col, VEC)])
                return carry
            lax.fori_loop(0, (CR * CC) // (VEC * UNROLL), kbody, 0)

        def step(c, b):
            wait_in(c, b)
            compute(b)
            issue_out(c, b)


        for b in range(NBUF):
            issue_in(b, b)

        for c in range(NBUF):
            step(c, c)
            if c >= 1:
                wait_out(c - 1, (c - 1) % NBUF)
                if c - 1 + NBUF < NCHUNK:
                    issue_in(c - 1 + NBUF, (c - 1) % NBUF)

        NSTEADY = (NCHUNK - 2 * NBUF + 1) // NBUF

        def gbody(g, carry):
            for b in range(NBUF):
                c = g * NBUF + b
                step(c, b)
                wait_out(c - 1, (b + NBUF - 1) % NBUF)
                issue_in(c - 1 + NBUF, (b + NBUF - 1) % NBUF)
            return carry

        lax.fori_loop(1, NSTEADY + 1, gbody, 0)

        for c in range((NSTEADY + 1) * NBUF, NCHUNK):
            step(c, c % NBUF)
            wait_out(c - 1, (c - 1) % NBUF)
            if c - 1 + NBUF < NCHUNK:
                issue_in(c - 1 + NBUF, (c - 1) % NBUF)

        wait_out(NCHUNK - 1, (NCHUNK - 1) % NBUF)

    return sc_add(x2, pos_table).reshape(B, S, D)


# speedup vs baseline: 1.5952x; 1.0277x over previous
import functools

import jax
import jax.numpy as jnp
from jax import lax
from jax.experimental import pallas as pl
from jax.experimental.pallas import tpu as pltpu
from jax.experimental.pallas import tpu_sc as plsc

B, S, D = 4, 8192, 1024
NC, NS = 2, 16
NW = NC * NS
ROWS = B * S
RPW = ROWS // NW
CR = 64
CC = 128
NCC = D // CC
NCHUNK = (RPW // CR) * NCC
NBUF = 4
NG = NCHUNK // NBUF
VEC = 16
UNROLL = 16


def kernel(inputs, pos_table):
    x2 = inputs.reshape(ROWS, D)
    mesh = plsc.VectorSubcoreMesh(core_axis_name="c", subcore_axis_name="s")

    @functools.partial(
        pl.kernel,
        mesh=mesh,
        out_type=jax.ShapeDtypeStruct((ROWS, D), jnp.float32),
        scratch_types=(
            [pltpu.VMEM((CR, CC), jnp.float32) for _ in range(2 * NBUF)]
            + [pltpu.SemaphoreType.DMA((NBUF,)), pltpu.SemaphoreType.DMA((NBUF,))]
        ),
    )
    def sc_add(x_hbm, t_hbm, o_hbm, *scratch):
        xb = scratch[:NBUF]
        ob = scratch[NBUF:2 * NBUF]
        in_sem, out_sem = scratch[2 * NBUF], scratch[2 * NBUF + 1]

        wid = lax.axis_index("s") * NC + lax.axis_index("c")
        row0 = wid * RPW
        trow0 = row0 % S

        def offs(c):
            r = (c // NCC) * CR
            col = (c % NCC) * CC
            return r, col

        def issue_in(c, b):
            r, col = offs(c)
            pltpu.async_copy(x_hbm.at[pl.ds(row0 + r, CR), pl.ds(col, CC)],
                             xb[b], in_sem.at[b])
            pltpu.async_copy(t_hbm.at[pl.ds(trow0 + r, CR), pl.ds(col, CC)],
                             ob[b], in_sem.at[b])

        def wait_in(c, b):
            r, col = offs(c)
            pltpu.make_async_copy(x_hbm.at[pl.ds(row0 + r, CR), pl.ds(col, CC)],
                                  xb[b], in_sem.at[b]).wait()
            pltpu.make_async_copy(t_hbm.at[pl.ds(trow0 + r, CR), pl.ds(col, CC)],
                                  ob[b], in_sem.at[b]).wait()

        def issue_out(c, b):
            r, col = offs(c)
            pltpu.async_copy(ob[b], o_hbm.at[pl.ds(row0 + r, CR), pl.ds(col, CC)],
                             out_sem.at[b])

        def wait_out(c, b):
            r, col = offs(c)
            pltpu.make_async_copy(ob[b], o_hbm.at[pl.ds(row0 + r, CR), pl.ds(col, CC)],
                                  out_sem.at[b]).wait()

        def compute(b):
            def kbody(k, carry):
                for u in range(UNROLL):
                    idx = k * UNROLL + u
                    r = idx // (CC // VEC)
                    col = (idx % (CC // VEC)) * VEC
                    plsc.addupdate(ob[b].at[r, pl.ds(col, VEC)],
                                   xb[b][r, pl.ds(col, VEC)])
                return carry
            lax.fori_loop(0, (CR * CC) // (VEC * UNROLL), kbody, 0)

        def step(c, b):
            wait_in(c, b)
            compute(b)
            issue_out(c, b)


        for b in range(NBUF):
            issue_in(b, b)

        for c in range(NBUF):
            step(c, c)
            if c >= 1:
                wait_out(c - 1, (c - 1) % NBUF)
                if c - 1 + NBUF < NCHUNK:
                    issue_in(c - 1 + NBUF, (c - 1) % NBUF)

        def gbody(g, carry):
            for b in range(NBUF):
                c = g * NBUF + b
                step(c, b)
                wait_out(c - 1, (b + NBUF - 1) % NBUF)
                issue_in(c - 1 + NBUF, (b + NBUF - 1) % NBUF)
            return carry

        lax.fori_loop(1, NG - 1, gbody, 0)

        for b in range(NBUF):
            c = NCHUNK - NBUF + b
            step(c, b)
            wait_out(c - 1, (c - 1) % NBUF)
            if c - 1 + NBUF < NCHUNK:
                issue_in(c - 1 + NBUF, (c - 1) % NBUF)

        wait_out(NCHUNK - 1, (NCHUNK - 1) % NBUF)

    return sc_add(x2, pos_table).reshape(B, S, D)
